# baseline (speedup 1.0000x reference)
import jax
import jax.numpy as jnp
from jax.experimental import pallas as pl
from jax.experimental.pallas import tpu as pltpu

_BN = 200


def _body(nei_ref, h_ref, wm_ref, bm_ref, w1_ref, w2_ref, b_ref, out_ref):
    bn, deg = h_ref.shape[0], nei_ref.shape[0] // h_ref.shape[0]
    d = nei_ref.shape[1]
    x = jnp.dot(nei_ref[...], wm_ref[...], preferred_element_type=jnp.float32)
    x = jnp.maximum(x + bm_ref[...], 0.0)
    m = jnp.max(x.reshape(bn, deg, d), axis=1)
    out = jnp.dot(m, w1_ref[...], preferred_element_type=jnp.float32)
    out = out + jnp.dot(h_ref[...], w2_ref[...], preferred_element_type=jnp.float32)
    out_ref[...] = out + b_ref[...]


def kernel(h, nei, W_mlp, b_mlp, W, b):
    n, d = h.shape
    deg = nei.shape[1]
    out_dim = W.shape[0]
    bn = _BN
    nei2 = nei.reshape(n * deg, d)
    wm = W_mlp.T
    w1 = W[:, :d].T
    w2 = W[:, d:].T
    bm = b_mlp.reshape(1, d)
    bb = b.reshape(1, out_dim)
    return pl.pallas_call(
        _body,
        grid=(n // bn,),
        in_specs=[
            pl.BlockSpec((bn * deg, d), lambda i: (i, 0)),
            pl.BlockSpec((bn, d), lambda i: (i, 0)),
            pl.BlockSpec((d, d), lambda i: (0, 0)),
            pl.BlockSpec((1, d), lambda i: (0, 0)),
            pl.BlockSpec((d, out_dim), lambda i: (0, 0)),
            pl.BlockSpec((d, out_dim), lambda i: (0, 0)),
            pl.BlockSpec((1, out_dim), lambda i: (0, 0)),
        ],
        out_specs=pl.BlockSpec((bn, out_dim), lambda i: (i, 0)),
        out_shape=jax.ShapeDtypeStruct((n, out_dim), jnp.float32),
        compiler_params=pltpu.CompilerParams(
            dimension_semantics=("parallel",),
        ),
    )(nei2, h, wm, bm, w1, w2, bb)


# speedup vs baseline: 1.3358x; 1.3358x over previous
import jax
import jax.numpy as jnp
from jax.experimental import pallas as pl
from jax.experimental.pallas import tpu as pltpu

_BN = 1000


def _body(nei_ref, h_ref, wm_ref, bm_ref, w1_ref, w2_ref, b_ref, out_ref):
    bn, deg = h_ref.shape[0], nei_ref.shape[0] // h_ref.shape[0]
    d = nei_ref.shape[1]
    x = jnp.dot(nei_ref[...], wm_ref[...], preferred_element_type=jnp.float32)
    x = jnp.maximum(x + bm_ref[...], 0.0)
    m = jnp.max(x.reshape(bn, deg, d), axis=1)
    out = jnp.dot(m, w1_ref[...], preferred_element_type=jnp.float32)
    out = out + jnp.dot(h_ref[...], w2_ref[...], preferred_element_type=jnp.float32)
    out_ref[...] = out + b_ref[...]


def kernel(h, nei, W_mlp, b_mlp, W, b):
    n, d = h.shape
    deg = nei.shape[1]
    out_dim = W.shape[0]
    bn = _BN
    nei2 = nei.reshape(n * deg, d)
    wm = W_mlp.T
    w1 = W[:, :d].T
    w2 = W[:, d:].T
    bm = b_mlp.reshape(1, d)
    bb = b.reshape(1, out_dim)
    return pl.pallas_call(
        _body,
        grid=(n // bn,),
        in_specs=[
            pl.BlockSpec((bn * deg, d), lambda i: (i, 0)),
            pl.BlockSpec((bn, d), lambda i: (i, 0)),
            pl.BlockSpec((d, d), lambda i: (0, 0)),
            pl.BlockSpec((1, d), lambda i: (0, 0)),
            pl.BlockSpec((d, out_dim), lambda i: (0, 0)),
            pl.BlockSpec((d, out_dim), lambda i: (0, 0)),
            pl.BlockSpec((1, out_dim), lambda i: (0, 0)),
        ],
        out_specs=pl.BlockSpec((bn, out_dim), lambda i: (i, 0)),
        out_shape=jax.ShapeDtypeStruct((n, out_dim), jnp.float32),
        compiler_params=pltpu.CompilerParams(
            dimension_semantics=("parallel",),
        ),
    )(nei2, h, wm, bm, w1, w2, bb)


# speedup vs baseline: 1.3515x; 1.0118x over previous
import jax
import jax.numpy as jnp
from jax.experimental import pallas as pl
from jax.experimental.pallas import tpu as pltpu

_BN = 1000


def _body(nei_ref, h_ref, wm_ref, bm_ref, w1_ref, w2_ref, b_ref, out_ref):
    bn, deg = h_ref.shape[0], nei_ref.shape[0] // h_ref.shape[0]
    d = nei_ref.shape[1]
    x = jnp.dot(nei_ref[...].astype(jnp.bfloat16), wm_ref[...].astype(jnp.bfloat16),
                preferred_element_type=jnp.float32)
    x = jnp.maximum(x + bm_ref[...], 0.0)
    m = jnp.max(x.reshape(bn, deg, d), axis=1)
    out = jnp.dot(m, w1_ref[...], preferred_element_type=jnp.float32)
    out = out + jnp.dot(h_ref[...], w2_ref[...], preferred_element_type=jnp.float32)
    out_ref[...] = out + b_ref[...]


def kernel(h, nei, W_mlp, b_mlp, W, b):
    n, d = h.shape
    deg = nei.shape[1]
    out_dim = W.shape[0]
    bn = _BN
    nei2 = nei.reshape(n * deg, d)
    wm = W_mlp.T
    w1 = W[:, :d].T
    w2 = W[:, d:].T
    bm = b_mlp.reshape(1, d)
    bb = b.reshape(1, out_dim)
    return pl.pallas_call(
        _body,
        grid=(n // bn,),
        in_specs=[
            pl.BlockSpec((bn * deg, d), lambda i: (i, 0)),
            pl.BlockSpec((bn, d), lambda i: (i, 0)),
            pl.BlockSpec((d, d), lambda i: (0, 0)),
            pl.BlockSpec((1, d), lambda i: (0, 0)),
            pl.BlockSpec((d, out_dim), lambda i: (0, 0)),
            pl.BlockSpec((d, out_dim), lambda i: (0, 0)),
            pl.BlockSpec((1, out_dim), lambda i: (0, 0)),
        ],
        out_specs=pl.BlockSpec((bn, out_dim), lambda i: (i, 0)),
        out_shape=jax.ShapeDtypeStruct((n, out_dim), jnp.float32),
        compiler_params=pltpu.CompilerParams(
            dimension_semantics=("parallel",),
        ),
    )(nei2, h, wm, bm, w1, w2, bb)


# speedup vs baseline: 1.3644x; 1.0095x over previous
import jax
import jax.numpy as jnp
from jax.experimental import pallas as pl
from jax.experimental.pallas import tpu as pltpu

_BN = 1000


def _body(nei_ref, h_ref, wm_ref, bm_ref, w1_ref, w2_ref, b_ref, out_ref):
    bn, deg = h_ref.shape[0], nei_ref.shape[0] // h_ref.shape[0]
    d = nei_ref.shape[1]
    x = jnp.dot(nei_ref[...].astype(jnp.bfloat16), wm_ref[...].astype(jnp.bfloat16),
                preferred_element_type=jnp.float32)
    m = jnp.maximum(jnp.max(x.reshape(bn, deg, d), axis=1) + bm_ref[...], 0.0)
    out = jnp.dot(m, w1_ref[...], preferred_element_type=jnp.float32)
    out = out + jnp.dot(h_ref[...], w2_ref[...], preferred_element_type=jnp.float32)
    out_ref[...] = out + b_ref[...]


def kernel(h, nei, W_mlp, b_mlp, W, b):
    n, d = h.shape
    deg = nei.shape[1]
    out_dim = W.shape[0]
    bn = _BN
    nei2 = nei.reshape(n * deg, d)
    wm = W_mlp.T
    w1 = W[:, :d].T
    w2 = W[:, d:].T
    bm = b_mlp.reshape(1, d)
    bb = b.reshape(1, out_dim)
    return pl.pallas_call(
        _body,
        grid=(n // bn,),
        in_specs=[
            pl.BlockSpec((bn * deg, d), lambda i: (i, 0)),
            pl.BlockSpec((bn, d), lambda i: (i, 0)),
            pl.BlockSpec((d, d), lambda i: (0, 0)),
            pl.BlockSpec((1, d), lambda i: (0, 0)),
            pl.BlockSpec((d, out_dim), lambda i: (0, 0)),
            pl.BlockSpec((d, out_dim), lambda i: (0, 0)),
            pl.BlockSpec((1, out_dim), lambda i: (0, 0)),
        ],
        out_specs=pl.BlockSpec((bn, out_dim), lambda i: (i, 0)),
        out_shape=jax.ShapeDtypeStruct((n, out_dim), jnp.float32),
        compiler_params=pltpu.CompilerParams(
            dimension_semantics=("parallel",),
        ),
    )(nei2, h, wm, bm, w1, w2, bb)
